# Initial kernel scaffold; baseline (speedup 1.0000x reference)
import jax
import jax.numpy as jnp
from jax.experimental import pallas as pl


def kernel(nodes, pos_neighbors, neg_neighbors, node_embeddings):
    raise NotImplementedError("write your pallas kernel here")


# speedup vs baseline: 2.3287x; 2.3287x over previous
import functools

import jax
import jax.numpy as jnp
from jax import lax
from jax.experimental import pallas as pl
from jax.experimental.pallas import tpu as pltpu
from jax.experimental.pallas import tpu_sc as plsc

D = 64
K = 5
L = 16

_info = plsc.get_sparse_core_info()
NC, NS = _info.num_cores, _info.num_subcores
NW = NC * NS


def _sc_scores(batch: int):
    bpw = batch // NW
    chunk = 128
    nchunk = bpw // chunk
    mesh = plsc.VectorSubcoreMesh(core_axis_name="c", subcore_axis_name="s")

    @functools.partial(
        pl.kernel,
        out_type=jax.ShapeDtypeStruct((1 + K, batch), jnp.float32),
        mesh=mesh,
        compiler_params=pltpu.CompilerParams(
            needs_layout_passes=False, use_tc_tiling_on_sc=False
        ),
        scratch_types=[
            pltpu.VMEM((chunk,), jnp.int32),
            pltpu.VMEM((chunk,), jnp.int32),
            pltpu.VMEM((chunk * K,), jnp.int32),
            pltpu.VMEM((chunk, D), jnp.float32),
            pltpu.VMEM((chunk, D), jnp.float32),
            pltpu.VMEM((chunk * K, D), jnp.float32),
            pltpu.VMEM((1 + K, chunk), jnp.float32),
            pltpu.SemaphoreType.DMA,
        ],
    )
    def k(nodes_hbm, pos_hbm, negf_hbm, table_hbm, out_hbm,
          idx_n, idx_p, idx_g, rows_n, rows_p, rows_g, scores, sem):
        wid = lax.axis_index("s") * NC + lax.axis_index("c")
        base0 = wid * bpw
        iot = lax.iota(jnp.int32, L)

        def chunk_body(c, _):
            base = pl.multiple_of(base0 + c * chunk, chunk)
            pltpu.sync_copy(nodes_hbm.at[pl.ds(base, chunk)], idx_n)
            pltpu.sync_copy(pos_hbm.at[pl.ds(base, chunk)], idx_p)
            pltpu.sync_copy(negf_hbm.at[pl.ds(base * K, chunk * K)], idx_g)
            cp_n = pltpu.async_copy(table_hbm.at[idx_n], rows_n, sem)
            cp_p = pltpu.async_copy(table_hbm.at[idx_p], rows_p, sem)
            cp_g = pltpu.async_copy(table_hbm.at[idx_g], rows_g, sem)
            cp_n.wait()
            cp_p.wait()
            cp_g.wait()

            def group_body(g, _):
                rb = g * L + iot
                rbk = rb * K
                accp = jnp.zeros((L,), jnp.float32)
                accn = [jnp.zeros((L,), jnp.float32) for _ in range(K)]
                for d in range(D):
                    col = jnp.full((L,), d, jnp.int32)
                    nd = plsc.load_gather(rows_n, [rb, col])
                    pd = plsc.load_gather(rows_p, [rb, col])
                    accp = accp + nd * pd
                    for kk in range(K):
                        gd = plsc.load_gather(rows_g, [rbk + kk, col])
                        accn[kk] = accn[kk] + nd * gd
                scores[0, pl.ds(g * L, L)] = accp
                for kk in range(K):
                    scores[1 + kk, pl.ds(g * L, L)] = accn[kk]
                return 0

            lax.fori_loop(0, chunk // L, group_body, 0)
            pltpu.sync_copy(scores, out_hbm.at[:, pl.ds(base, chunk)])
            return 0

        lax.fori_loop(0, nchunk, chunk_body, 0)

    return k


def _tc_losses(batch: int):
    def body(s_ref, pos_out, neg_out):
        s = s_ref[...]
        sp = jnp.log1p(jnp.exp(-jnp.abs(s)))
        pos = jnp.maximum(-s[0:1, :], 0.0) + sp[0:1, :]
        neg = jnp.maximum(s[1:, :], 0.0) + sp[1:, :]
        pos_out[0, 0] = jnp.sum(pos) / batch
        neg_out[0, 0] = jnp.sum(neg) / (batch * K)

    return pl.pallas_call(
        body,
        out_shape=[
            jax.ShapeDtypeStruct((1, 1), jnp.float32),
            jax.ShapeDtypeStruct((1, 1), jnp.float32),
        ],
        out_specs=[
            pl.BlockSpec(memory_space=pltpu.SMEM),
            pl.BlockSpec(memory_space=pltpu.SMEM),
        ],
    )


def kernel(nodes, pos_neighbors, neg_neighbors, node_embeddings):
    batch = nodes.shape[0]
    negf = neg_neighbors.reshape(-1)
    scores = _sc_scores(batch)(nodes, pos_neighbors, negf, node_embeddings)
    pos_loss, neg_loss = _tc_losses(batch)(scores)
    return (pos_loss[0, 0], neg_loss[0, 0])


# speedup vs baseline: 2.7667x; 1.1881x over previous
import functools

import jax
import jax.numpy as jnp
from jax import lax
from jax.experimental import pallas as pl
from jax.experimental.pallas import tpu as pltpu
from jax.experimental.pallas import tpu_sc as plsc

D = 64
K = 5
L = 16

_info = plsc.get_sparse_core_info()
NC, NS = _info.num_cores, _info.num_subcores
NW = NC * NS


def _sc_scores(batch: int):
    bpw = batch // NW
    chunk = 128
    nchunk = bpw // chunk
    rows_per_chunk = chunk * (2 + K)
    mesh = plsc.VectorSubcoreMesh(core_axis_name="c", subcore_axis_name="s")

    @functools.partial(
        pl.kernel,
        out_type=jax.ShapeDtypeStruct((1 + K, batch), jnp.float32),
        mesh=mesh,
        compiler_params=pltpu.CompilerParams(
            needs_layout_passes=False, use_tc_tiling_on_sc=False
        ),
        scratch_types=[
            pltpu.VMEM((nchunk, rows_per_chunk), jnp.int32),
            pltpu.VMEM((rows_per_chunk, D), jnp.float32),
            pltpu.VMEM((rows_per_chunk, D), jnp.float32),
            pltpu.VMEM((nchunk, 1 + K, chunk), jnp.float32),
            pltpu.SemaphoreType.DMA,
            pltpu.SemaphoreType.DMA,
            pltpu.SemaphoreType.DMA,
            pltpu.SemaphoreType.DMA,
        ],
    )
    def k(nodes_hbm, pos_hbm, negf_hbm, table_hbm, out_hbm,
          idx_all, rows_a, rows_b, scores, sem_i, sem_e, sem_o, sem_w):
        wid = lax.axis_index("s") * NC + lax.axis_index("c")
        base0 = wid * bpw
        iot = lax.iota(jnp.int32, L)
        bufs = (rows_a, rows_b)
        ssems = (sem_e, sem_o)

        idx_cps = []
        for c in range(nchunk):
            bc = base0 + c * chunk
            idx_cps.append(pltpu.async_copy(
                nodes_hbm.at[pl.ds(bc, chunk)],
                idx_all.at[c, pl.ds(0, chunk)], sem_i))
            idx_cps.append(pltpu.async_copy(
                pos_hbm.at[pl.ds(bc, chunk)],
                idx_all.at[c, pl.ds(chunk, chunk)], sem_i))
            idx_cps.append(pltpu.async_copy(
                negf_hbm.at[pl.ds(bc * K, chunk * K)],
                idx_all.at[c, pl.ds(2 * chunk, chunk * K)], sem_i))
        for cp in idx_cps:
            cp.wait()

        streams = [None] * nchunk
        streams[0] = pltpu.async_copy(
            table_hbm.at[idx_all.at[0]], bufs[0], ssems[0])

        out_cps = []
        for c in range(nchunk):
            if c + 1 < nchunk:
                streams[c + 1] = pltpu.async_copy(
                    table_hbm.at[idx_all.at[c + 1]],
                    bufs[(c + 1) % 2], ssems[(c + 1) % 2])
            streams[c].wait()
            rows = bufs[c % 2]

            def group_body(g, _, rows=rows, c=c):
                rb = g * L + iot
                rp = rb + chunk
                rg = 2 * chunk + rb * K
                accs = [jnp.zeros((L,), jnp.float32) for _ in range(1 + K)]

                def d_body(d4, accs):
                    accs = list(accs)
                    for dd in range(4):
                        col = jnp.bitwise_and(iot + (d4 * 4 + dd), D - 1)
                        nd = plsc.load_gather(rows, [rb, col])
                        pd = plsc.load_gather(rows, [rp, col])
                        accs[0] = accs[0] + nd * pd
                        for kk in range(K):
                            gd = plsc.load_gather(rows, [rg + kk, col])
                            accs[1 + kk] = accs[1 + kk] + nd * gd
                    return tuple(accs)

                accs = lax.fori_loop(0, D // 4, d_body, tuple(accs))
                for s in range(1 + K):
                    scores[c, s, pl.ds(g * L, L)] = accs[s]
                return 0

            lax.fori_loop(0, chunk // L, group_body, 0)
            out_cps.append(pltpu.async_copy(
                scores.at[c],
                out_hbm.at[:, pl.ds(base0 + c * chunk, chunk)], sem_w))
        for cp in out_cps:
            cp.wait()

    return k


def _tc_losses(batch: int):
    def body(s_ref, pos_out, neg_out):
        s = s_ref[...]
        sp = jnp.log1p(jnp.exp(-jnp.abs(s)))
        pos = jnp.maximum(-s[0:1, :], 0.0) + sp[0:1, :]
        neg = jnp.maximum(s[1:, :], 0.0) + sp[1:, :]
        pos_out[0, 0] = jnp.sum(pos) / batch
        neg_out[0, 0] = jnp.sum(neg) / (batch * K)

    return pl.pallas_call(
        body,
        out_shape=[
            jax.ShapeDtypeStruct((1, 1), jnp.float32),
            jax.ShapeDtypeStruct((1, 1), jnp.float32),
        ],
        out_specs=[
            pl.BlockSpec(memory_space=pltpu.SMEM),
            pl.BlockSpec(memory_space=pltpu.SMEM),
        ],
    )


def kernel(nodes, pos_neighbors, neg_neighbors, node_embeddings):
    batch = nodes.shape[0]
    negf = neg_neighbors.reshape(-1)
    scores = _sc_scores(batch)(nodes, pos_neighbors, negf, node_embeddings)
    pos_loss, neg_loss = _tc_losses(batch)(scores)
    return (pos_loss[0, 0], neg_loss[0, 0])
